# baseline (speedup 1.0000x reference)
import functools

import jax
import jax.numpy as jnp
from jax import lax
from jax.experimental import pallas as pl
from jax.experimental.pallas import tpu as pltpu
from jax.experimental.pallas import tpu_sc as plsc

NUM_CORES = 2
NUM_SUBCORES = 16
NUM_WORKERS = NUM_CORES * NUM_SUBCORES

CHUNK_ROWS = 16
NBUF = 3
SC_BATCH = 2
TC_BLK = 256


def _sc_body(chunks_per_worker,
             w_hbm, sp_hbm, out_hbm,
             sp_v, buf0, buf1, buf2,
             li0, li1, li2, so0, so1, so2):
    core = lax.axis_index("c")
    sub = lax.axis_index("s")
    wid = sub * NUM_CORES + core
    rows_per_worker = chunks_per_worker * CHUNK_ROWS
    base = wid * rows_per_worker

    bufs = [buf0, buf1, buf2]
    lsems = [li0, li1, li2]
    ssems = [so0, so1, so2]

    pltpu.sync_copy(sp_hbm, sp_v)
    start = pl.multiple_of(sp_v[...][0], 8)

    def load(c):
        return pltpu.async_copy(
            w_hbm.at[pl.ds(start + base + c * CHUNK_ROWS, CHUNK_ROWS)],
            bufs[c % NBUF], lsems[c % NBUF])

    def store(c):
        return [pltpu.async_copy(
            bufs[c % NBUF],
            out_hbm.at[b, pl.ds(base + c * CHUNK_ROWS, CHUNK_ROWS)],
            ssems[c % NBUF]) for b in range(SC_BATCH)]

    nch = chunks_per_worker
    loads = [None] * nch
    stores = [None] * nch
    store_waited = [False] * nch
    for c in range(min(NBUF - 1, nch)):
        loads[c] = load(c)
    for c in range(nch):
        if c + NBUF - 1 < nch:
            prev = c - 1
            if prev >= 0:
                for cp in stores[prev]:
                    cp.wait()
                store_waited[prev] = True
            loads[c + NBUF - 1] = load(c + NBUF - 1)
        loads[c].wait()
        stores[c] = store(c)
    for c in range(nch):
        if not store_waited[c]:
            for cp in stores[c]:
                cp.wait()


def _tc_brd(sref, w_ref, out_ref):
    out_ref[...] = jnp.broadcast_to(w_ref[...][None], out_ref.shape)


def kernel(tokens, start_pos, W_pos):
    batch, seq_len = tokens.shape
    d_model = W_pos.shape[-1]
    chunks_per_worker = seq_len // (NUM_WORKERS * CHUNK_ROWS)
    tc_batch = batch - SC_BATCH

    sp_arr = jnp.full((16,), start_pos, dtype=jnp.int32)

    sc_mesh = plsc.VectorSubcoreMesh(
        core_axis_name="c", subcore_axis_name="s",
        num_cores=NUM_CORES, num_subcores=NUM_SUBCORES)

    sc_out = pl.kernel(
        functools.partial(_sc_body, chunks_per_worker),
        out_type=jax.ShapeDtypeStruct((SC_BATCH, seq_len, d_model),
                                      W_pos.dtype),
        mesh=sc_mesh,
        scratch_types=[
            pltpu.VMEM((16,), jnp.int32),
            pltpu.VMEM((CHUNK_ROWS, d_model), W_pos.dtype),
            pltpu.VMEM((CHUNK_ROWS, d_model), W_pos.dtype),
            pltpu.VMEM((CHUNK_ROWS, d_model), W_pos.dtype),
            pltpu.SemaphoreType.DMA, pltpu.SemaphoreType.DMA,
            pltpu.SemaphoreType.DMA, pltpu.SemaphoreType.DMA,
            pltpu.SemaphoreType.DMA, pltpu.SemaphoreType.DMA,
        ],
    )(W_pos, sp_arr)

    sp1 = jnp.full((1,), start_pos, dtype=jnp.int32)
    tc_out = pl.pallas_call(
        _tc_brd,
        grid_spec=pltpu.PrefetchScalarGridSpec(
            num_scalar_prefetch=1,
            grid=(seq_len // TC_BLK,),
            in_specs=[pl.BlockSpec((TC_BLK, d_model),
                                   lambda i, s: (s[0] // TC_BLK + i, 0))],
            out_specs=pl.BlockSpec((tc_batch, TC_BLK, d_model),
                                   lambda i, s: (0, i, 0)),
        ),
        out_shape=jax.ShapeDtypeStruct((tc_batch, seq_len, d_model),
                                       W_pos.dtype),
    )(sp1, W_pos)

    return jnp.concatenate([sc_out, tc_out], axis=0)


# speedup vs baseline: 2.0808x; 2.0808x over previous
import functools

import jax
import jax.numpy as jnp
from jax import lax
from jax.experimental import pallas as pl
from jax.experimental.pallas import tpu as pltpu
from jax.experimental.pallas import tpu_sc as plsc

NUM_CORES = 2
NUM_SUBCORES = 16
NUM_WORKERS = NUM_CORES * NUM_SUBCORES

CHUNK_ROWS = 16


def _pipeline(nch, load, store):
    loads = [None] * nch
    stores = [None] * nch
    waited = [False] * nch
    loads[0] = load(0)
    for c in range(nch):
        if c + 1 < nch:
            if c - 1 >= 0:
                for cp in stores[c - 1]:
                    cp.wait()
                waited[c - 1] = True
            loads[c + 1] = load(c + 1)
        loads[c].wait()
        stores[c] = store(c)
    return stores, waited


def _pos_embed_body(batch, chunks_per_worker,
                    w_hbm, sp_hbm, out_hbm,
                    sp_v, bufa0, bufa1, shared,
                    la0, la1, sa0, sa1, lb0, lb1, sb0, sb1):
    core = lax.axis_index("c")
    sub = lax.axis_index("s")
    wid = sub * NUM_CORES + core
    rows_per_worker = chunks_per_worker * CHUNK_ROWS
    base = wid * rows_per_worker

    pltpu.sync_copy(sp_hbm, sp_v)
    start = pl.multiple_of(sp_v[...][0], 8)

    bufsa = [bufa0, bufa1]
    lsa = [la0, la1]
    ssa = [sa0, sa1]
    lsb = [lb0, lb1]
    ssb = [sb0, sb1]

    na = (chunks_per_worker + 1) // 2
    nb = chunks_per_worker // 2

    def row0_a(i):
        return base + (2 * i) * CHUNK_ROWS

    def row0_b(i):
        return base + (2 * i + 1) * CHUNK_ROWS

    def load_a(i):
        return pltpu.async_copy(
            w_hbm.at[pl.ds(start + row0_a(i), CHUNK_ROWS)],
            bufsa[i % 2], lsa[i % 2])

    def store_a(i):
        return [pltpu.async_copy(
            bufsa[i % 2],
            out_hbm.at[b, pl.ds(row0_a(i), CHUNK_ROWS)],
            ssa[i % 2]) for b in range(batch)]

    def load_b(i):
        return pltpu.async_copy(
            w_hbm.at[pl.ds(start + row0_b(i), CHUNK_ROWS)],
            shared.at[sub], lsb[i % 2])

    def store_b(i):
        return [pltpu.async_copy(
            shared.at[sub],
            out_hbm.at[b, pl.ds(row0_b(i), CHUNK_ROWS)],
            ssb[i % 2]) for b in range(batch)]

    lda = [None] * na
    ldb = [None] * nb
    sta = [None] * na
    stb = [None] * nb
    wa = [False] * na
    wb = [False] * nb
    lda[0] = load_a(0)
    if nb:
        ldb[0] = load_b(0)
    for i in range(max(na, nb)):
        if i < na:
            if i + 1 < na:
                if i - 1 >= 0:
                    for cp in sta[i - 1]:
                        cp.wait()
                    wa[i - 1] = True
                lda[i + 1] = load_a(i + 1)
            lda[i].wait()
            sta[i] = store_a(i)
        if i < nb:
            ldb[i].wait()
            stb[i] = store_b(i)
            if i + 1 < nb:
                for cp in stb[i]:
                    cp.wait()
                wb[i] = True
                ldb[i + 1] = load_b(i + 1)
    for i in range(na):
        if not wa[i]:
            for cp in sta[i]:
                cp.wait()
    for i in range(nb):
        if not wb[i]:
            for cp in stb[i]:
                cp.wait()


def kernel(tokens, start_pos, W_pos):
    batch, seq_len = tokens.shape
    d_model = W_pos.shape[-1]
    assert seq_len % (NUM_WORKERS * CHUNK_ROWS) == 0
    chunks_per_worker = seq_len // (NUM_WORKERS * CHUNK_ROWS)

    sp_arr = jnp.full((16,), start_pos, dtype=jnp.int32)

    mesh = plsc.VectorSubcoreMesh(
        core_axis_name="c", subcore_axis_name="s",
        num_cores=NUM_CORES, num_subcores=NUM_SUBCORES)

    body = functools.partial(_pos_embed_body, batch, chunks_per_worker)

    out = pl.kernel(
        body,
        out_type=jax.ShapeDtypeStruct((batch, seq_len, d_model), W_pos.dtype),
        mesh=mesh,
        scratch_types=[
            pltpu.VMEM((16,), jnp.int32),
            pltpu.VMEM((CHUNK_ROWS, d_model), W_pos.dtype),
            pltpu.VMEM((CHUNK_ROWS, d_model), W_pos.dtype),
            pltpu.VMEM_SHARED((NUM_SUBCORES, CHUNK_ROWS, d_model),
                              W_pos.dtype),
            pltpu.SemaphoreType.DMA, pltpu.SemaphoreType.DMA,
            pltpu.SemaphoreType.DMA, pltpu.SemaphoreType.DMA,
            pltpu.SemaphoreType.DMA, pltpu.SemaphoreType.DMA,
            pltpu.SemaphoreType.DMA, pltpu.SemaphoreType.DMA,
        ],
    )(W_pos, sp_arr)
    return out


# speedup vs baseline: 2.1200x; 1.0189x over previous
import functools

import jax
import jax.numpy as jnp
from jax import lax
from jax.experimental import pallas as pl
from jax.experimental.pallas import tpu as pltpu
from jax.experimental.pallas import tpu_sc as plsc

NUM_CORES = 2
NUM_SUBCORES = 16
NUM_WORKERS = NUM_CORES * NUM_SUBCORES

CHUNK_ROWS = 16
NBUF = 3


def _pos_embed_body(batch, chunks_per_worker,
                    w_hbm, sp_hbm, out_hbm,
                    sp_v, buf0, buf1, buf2,
                    li0, li1, li2, so0, so1, so2):
    core = lax.axis_index("c")
    sub = lax.axis_index("s")
    wid = sub * NUM_CORES + core
    rows_per_worker = chunks_per_worker * CHUNK_ROWS
    base = wid * rows_per_worker

    bufs = [buf0, buf1, buf2]
    lsems = [li0, li1, li2]
    ssems = [so0, so1, so2]

    pltpu.sync_copy(sp_hbm, sp_v)
    start = pl.multiple_of(sp_v[...][0], 8)

    def load(c):
        return pltpu.async_copy(
            w_hbm.at[pl.ds(start + base + c * CHUNK_ROWS, CHUNK_ROWS)],
            bufs[c % NBUF], lsems[c % NBUF])

    def store(c):
        return [pltpu.async_copy(
            bufs[c % NBUF],
            out_hbm.at[b, pl.ds(base + c * CHUNK_ROWS, CHUNK_ROWS)],
            ssems[c % NBUF]) for b in range(batch)]

    nch = chunks_per_worker
    loads = [None] * nch
    stores = [None] * nch
    store_waited = [False] * nch
    for c in range(min(NBUF - 1, nch)):
        loads[c] = load(c)
    for c in range(nch):
        if c + NBUF - 1 < nch:
            prev = c - 1
            if prev >= 0:
                for cp in stores[prev]:
                    cp.wait()
                store_waited[prev] = True
            loads[c + NBUF - 1] = load(c + NBUF - 1)
        loads[c].wait()
        stores[c] = store(c)
    for c in range(nch):
        if not store_waited[c]:
            for cp in stores[c]:
                cp.wait()


def kernel(tokens, start_pos, W_pos):
    batch, seq_len = tokens.shape
    d_model = W_pos.shape[-1]
    assert seq_len % (NUM_WORKERS * CHUNK_ROWS) == 0
    chunks_per_worker = seq_len // (NUM_WORKERS * CHUNK_ROWS)

    sp_arr = jnp.full((16,), start_pos, dtype=jnp.int32)

    mesh = plsc.VectorSubcoreMesh(
        core_axis_name="c", subcore_axis_name="s",
        num_cores=NUM_CORES, num_subcores=NUM_SUBCORES)

    body = functools.partial(_pos_embed_body, batch, chunks_per_worker)

    out = pl.kernel(
        body,
        out_type=jax.ShapeDtypeStruct((batch, seq_len, d_model), W_pos.dtype),
        mesh=mesh,
        scratch_types=[
            pltpu.VMEM((16,), jnp.int32),
            pltpu.VMEM((CHUNK_ROWS, d_model), W_pos.dtype),
            pltpu.VMEM((CHUNK_ROWS, d_model), W_pos.dtype),
            pltpu.VMEM((CHUNK_ROWS, d_model), W_pos.dtype),
            pltpu.SemaphoreType.DMA,
            pltpu.SemaphoreType.DMA,
            pltpu.SemaphoreType.DMA,
            pltpu.SemaphoreType.DMA,
            pltpu.SemaphoreType.DMA,
            pltpu.SemaphoreType.DMA,
        ],
    )(W_pos, sp_arr)
    return out


# speedup vs baseline: 2.1376x; 1.0083x over previous
import functools

import jax
import jax.numpy as jnp
from jax import lax
from jax.experimental import pallas as pl
from jax.experimental.pallas import tpu as pltpu
from jax.experimental.pallas import tpu_sc as plsc

NUM_CORES = 2
NUM_SUBCORES = 16
NUM_WORKERS = NUM_CORES * NUM_SUBCORES

CHUNK_ROWS = 16
NBUF = 3


def _pos_embed_body(batch, chunks_per_worker,
                    w_hbm, sp_hbm, out_hbm,
                    sp_v, buf0, buf1, buf2,
                    li0, li1, li2, so0, so1, so2):
    core = lax.axis_index("c")
    sub = lax.axis_index("s")
    wid = sub * NUM_CORES + core
    rows_per_worker = chunks_per_worker * CHUNK_ROWS
    base = wid * rows_per_worker

    bufs = [buf0, buf1, buf2]
    lsems = [li0, li1, li2]
    ssems = [so0, so1, so2]

    pltpu.sync_copy(sp_hbm, sp_v)
    start = pl.multiple_of(sp_v[...][0], 8)

    def load(c):
        return pltpu.async_copy(
            w_hbm.at[pl.ds(start + base + c * CHUNK_ROWS, CHUNK_ROWS)],
            bufs[c % NBUF], lsems[c % NBUF])

    def store(c):
        return [pltpu.async_copy(
            bufs[c % NBUF],
            out_hbm.at[b, pl.ds(base + c * CHUNK_ROWS, CHUNK_ROWS)],
            ssems[c % NBUF]) for b in range(batch)]

    nch = chunks_per_worker
    loads = [None] * nch
    stores = [None] * nch
    store_waited = [False] * nch
    for c in range(min(NBUF - 1, nch)):
        loads[c] = load(c)
    for c in range(nch):
        if c + NBUF - 1 < nch:
            prev = c - 1
            if prev >= 0:
                for cp in stores[prev]:
                    cp.wait()
                store_waited[prev] = True
            loads[c + NBUF - 1] = load(c + NBUF - 1)
        loads[c].wait()
        stores[c] = store(c)
    for c in range(nch):
        if not store_waited[c]:
            for cp in stores[c]:
                cp.wait()


def kernel(tokens, start_pos, W_pos):
    batch, seq_len = tokens.shape
    d_model = W_pos.shape[-1]
    assert seq_len % (NUM_WORKERS * CHUNK_ROWS) == 0
    chunks_per_worker = seq_len // (NUM_WORKERS * CHUNK_ROWS)

    sp_arr = jnp.full((16,), start_pos, dtype=jnp.int32)

    mesh = plsc.VectorSubcoreMesh(
        core_axis_name="c", subcore_axis_name="s",
        num_cores=NUM_CORES, num_subcores=NUM_SUBCORES)

    body = functools.partial(_pos_embed_body, batch, chunks_per_worker)

    out = pl.kernel(
        body,
        out_type=jax.ShapeDtypeStruct((batch, seq_len, d_model), W_pos.dtype),
        mesh=mesh,
        compiler_params=pltpu.CompilerParams(
            disable_bounds_checks=True,
            disable_semaphore_checks=True,
            skip_device_barrier=True,
        ),
        scratch_types=[
            pltpu.VMEM((16,), jnp.int32),
            pltpu.VMEM((CHUNK_ROWS, d_model), W_pos.dtype),
            pltpu.VMEM((CHUNK_ROWS, d_model), W_pos.dtype),
            pltpu.VMEM((CHUNK_ROWS, d_model), W_pos.dtype),
            pltpu.SemaphoreType.DMA,
            pltpu.SemaphoreType.DMA,
            pltpu.SemaphoreType.DMA,
            pltpu.SemaphoreType.DMA,
            pltpu.SemaphoreType.DMA,
            pltpu.SemaphoreType.DMA,
        ],
    )(W_pos, sp_arr)
    return out
